# baseline (speedup 1.0000x reference)
import jax
import jax.numpy as jnp
from jax.experimental import pallas as pl
from jax.experimental.pallas import tpu as pltpu

_B = 1024
_S = 4096
_I = 256
_BJ = 256
_NJ = _S // _BJ


def _esn_flat(inputs_ref, states_ref, win_ref, wres_hbm, out_hbm,
              wbuf, obuf, sb_ref, ib_ref, wsem, osem):
    def wres_copy(t, slot):
        return pltpu.make_async_copy(
            wres_hbm.at[:, pl.ds(t * _BJ, _BJ)], wbuf.at[slot], wsem.at[slot])

    def out_copy(t, slot):
        return pltpu.make_async_copy(
            obuf.at[slot], out_hbm.at[:, pl.ds(t * _BJ, _BJ)], osem.at[slot])

    wres_copy(0, 0).start()
    sb_ref[...] = states_ref[...].astype(jnp.bfloat16)
    ib_ref[...] = inputs_ref[...].astype(jnp.bfloat16)
    sb = sb_ref[...]
    ib = ib_ref[...]
    for t in range(_NJ):
        slot = t % 2
        if t + 1 < _NJ:
            wres_copy(t + 1, slot ^ 1).start()
        wres_copy(t, slot).wait()
        wb = wbuf[slot].astype(jnp.bfloat16)
        winb = win_ref[:, pl.ds(t * _BJ, _BJ)].astype(jnp.bfloat16)
        z = jnp.dot(sb, wb, preferred_element_type=jnp.float32)
        z = z + jnp.dot(ib, winb, preferred_element_type=jnp.float32)
        cand = jnp.tanh(z)
        sj = states_ref[:, pl.ds(t * _BJ, _BJ)]
        if t >= 2:
            out_copy(t - 2, slot).wait()
        obuf[slot] = sj + (cand - sj)
        out_copy(t, slot).start()
    out_copy(_NJ - 2, (_NJ - 2) % 2).wait()
    out_copy(_NJ - 1, (_NJ - 1) % 2).wait()


def kernel(inputs, states, Win, Wres):
    return pl.pallas_call(
        _esn_flat,
        grid=(1,),
        in_specs=[
            pl.BlockSpec((_B, _I), lambda i: (0, 0)),
            pl.BlockSpec((_B, _S), lambda i: (0, 0)),
            pl.BlockSpec((_I, _S), lambda i: (0, 0)),
            pl.BlockSpec(memory_space=pltpu.MemorySpace.HBM),
        ],
        out_specs=pl.BlockSpec(memory_space=pltpu.MemorySpace.HBM),
        out_shape=jax.ShapeDtypeStruct((_B, _S), jnp.float32),
        scratch_shapes=[
            pltpu.VMEM((2, _S, _BJ), jnp.float32),
            pltpu.VMEM((2, _B, _BJ), jnp.float32),
            pltpu.VMEM((_B, _S), jnp.bfloat16),
            pltpu.VMEM((_B, _I), jnp.bfloat16),
            pltpu.SemaphoreType.DMA((2,)),
            pltpu.SemaphoreType.DMA((2,)),
        ],
    )(inputs, states, Win, Wres)


# speedup vs baseline: 1.8024x; 1.8024x over previous
import jax
import jax.numpy as jnp
from jax.experimental import pallas as pl

_B = 1024
_S = 4096
_I = 256
_BJ = 512
_NJ = _S // _BJ


def _esn_tile(inputs_ref, states_ref, win_ref, wres_ref, out_ref):
    t = pl.program_id(0)
    z = jnp.dot(states_ref[...], wres_ref[...],
                preferred_element_type=jnp.float32)
    z = z + jnp.dot(inputs_ref[...], win_ref[...],
                    preferred_element_type=jnp.float32)
    cand = jnp.tanh(z)
    sj = states_ref[:, pl.ds(t * _BJ, _BJ)]
    out_ref[...] = sj + (cand - sj)


def kernel(inputs, states, Win, Wres):
    return pl.pallas_call(
        _esn_tile,
        grid=(_NJ,),
        in_specs=[
            pl.BlockSpec((_B, _I), lambda t: (0, 0)),
            pl.BlockSpec((_B, _S), lambda t: (0, 0)),
            pl.BlockSpec((_I, _BJ), lambda t: (0, t)),
            pl.BlockSpec((_S, _BJ), lambda t: (0, t),
                         pipeline_mode=pl.Buffered(buffer_count=2)),
        ],
        out_specs=pl.BlockSpec((_B, _BJ), lambda t: (0, t),
                               pipeline_mode=pl.Buffered(buffer_count=2)),
        out_shape=jax.ShapeDtypeStruct((_B, _S), jnp.float32),
    )(inputs, states, Win, Wres)


# speedup vs baseline: 1.8036x; 1.0007x over previous
import jax
import jax.numpy as jnp
from jax.experimental import pallas as pl

_B = 1024
_S = 4096
_I = 256
_BJ = 512
_NJ = _S // _BJ


def _esn_tile(inputs_ref, states_ref, win_ref, wres_ref, out_ref):
    z = jnp.dot(states_ref[...], wres_ref[...],
                preferred_element_type=jnp.float32)
    z = z + jnp.dot(inputs_ref[...], win_ref[...],
                    preferred_element_type=jnp.float32)
    out_ref[...] = jnp.tanh(z)


def kernel(inputs, states, Win, Wres):
    return pl.pallas_call(
        _esn_tile,
        grid=(_NJ,),
        in_specs=[
            pl.BlockSpec((_B, _I), lambda t: (0, 0)),
            pl.BlockSpec((_B, _S), lambda t: (0, 0)),
            pl.BlockSpec((_I, _BJ), lambda t: (0, t)),
            pl.BlockSpec((_S, _BJ), lambda t: (0, t),
                         pipeline_mode=pl.Buffered(buffer_count=2)),
        ],
        out_specs=pl.BlockSpec((_B, _BJ), lambda t: (0, t),
                               pipeline_mode=pl.Buffered(buffer_count=2)),
        out_shape=jax.ShapeDtypeStruct((_B, _S), jnp.float32),
    )(inputs, states, Win, Wres)
